# Initial kernel scaffold; baseline (speedup 1.0000x reference)
import jax
import jax.numpy as jnp
from jax.experimental import pallas as pl


def kernel(user, item, tag_index, user_e, tag_e, user_b, tag_b, item_q):
    raise NotImplementedError("write your pallas kernel here")


# speedup vs baseline: 19.1931x; 19.1931x over previous
import jax
import jax.numpy as jnp
from jax import lax
from jax.experimental import pallas as pl
from jax.experimental.pallas import tpu as pltpu
from jax.experimental.pallas import tpu_sc as plsc

B = 4096
EMB = 128
HIST = 50
NC = 2
NS = 16
NW = NC * NS
BPW = B // NW
LANES = 16
GROUPS = BPW // LANES
INV_HIST = 1.0 / HIST


def _sc_body(user_h, item_h, tag_t_h, user_e_h, tag_e_h, user_b_h,
             tag_b_h, item_q_h, out_h,
             idx_v, user_v, item_v, eidx_v, ebuf0, ebuf1, tbuf0, tbuf1,
             acc_v, urow_v, tb_v, ub_v, iq_v, out_v, shared_acc,
             sem_u, sem_g0, sem_g1, sem_t0, sem_t1, sem_s):
    c = lax.axis_index("c")
    s = lax.axis_index("s")
    wid = s * NC + c
    base = wid * BPW
    sbase = s * BPW

    pltpu.sync_copy(tag_t_h.at[:, pl.ds(base, BPW)], idx_v)
    pltpu.sync_copy(user_h.at[pl.ds(base, BPW)], user_v)
    pltpu.sync_copy(item_h.at[pl.ds(base, BPW)], item_v)

    d_ue = pltpu.async_copy(user_e_h.at[user_v], urow_v, sem_u)
    d_ub = pltpu.async_copy(user_b_h.at[user_v], ub_v, sem_u)
    d_iq = pltpu.async_copy(item_q_h.at[item_v], iq_v, sem_u)

    lanes = lax.iota(jnp.int32, LANES)
    zero16 = jnp.zeros((LANES,), jnp.float32)
    for g in range(GROUPS):
        eidx_v[pl.ds(g * LANES, LANES)] = lanes + (sbase + g * LANES)
        tb_v[pl.ds(g * LANES, LANES)] = zero16

    def tb_add(buf):
        for g in range(GROUPS):
            sl = pl.ds(g * LANES, LANES)
            tb_v[sl] = tb_v[sl] + buf[sl]

    d_g = pltpu.async_copy(tag_e_h.at[idx_v.at[0]], ebuf0, sem_g0)
    d_t = pltpu.async_copy(tag_b_h.at[idx_v.at[0]], tbuf0, sem_t0)
    d_g.wait()
    d_s = pltpu.async_copy(ebuf0, shared_acc.at[eidx_v], sem_s)
    d_t.wait()
    tb_add(tbuf0)
    d_s.wait()

    def pair(i, carry):
        l1 = 2 * i + 1
        l2 = 2 * i + 2
        dg1 = pltpu.async_copy(tag_e_h.at[idx_v.at[l1]], ebuf1, sem_g1)
        dt1 = pltpu.async_copy(tag_b_h.at[idx_v.at[l1]], tbuf1, sem_t1)
        dg0 = pltpu.async_copy(tag_e_h.at[idx_v.at[l2]], ebuf0, sem_g0)
        dt0 = pltpu.async_copy(tag_b_h.at[idx_v.at[l2]], tbuf0, sem_t0)
        dg1.wait()
        ds1 = pltpu.async_copy(ebuf1, shared_acc.at[eidx_v], sem_s, add=True)
        dt1.wait()
        tb_add(tbuf1)
        dg0.wait()
        ds0 = pltpu.async_copy(ebuf0, shared_acc.at[eidx_v], sem_s, add=True)
        dt0.wait()
        tb_add(tbuf0)
        ds1.wait()
        ds0.wait()
        return carry

    lax.fori_loop(0, (HIST - 2) // 2, pair, 0)

    d_g = pltpu.async_copy(tag_e_h.at[idx_v.at[HIST - 1]], ebuf1, sem_g1)
    d_t = pltpu.async_copy(tag_b_h.at[idx_v.at[HIST - 1]], tbuf1, sem_t1)
    d_g.wait()
    d_s = pltpu.async_copy(ebuf1, shared_acc.at[eidx_v], sem_s, add=True)
    d_t.wait()
    tb_add(tbuf1)
    d_s.wait()

    pltpu.sync_copy(shared_acc.at[pl.ds(sbase, BPW)], acc_v)
    d_ue.wait()
    d_ub.wait()
    d_iq.wait()

    def group(g, carry):
        rows = lanes + g * LANES

        def kstep(k, dot):
            col = jnp.zeros((LANES,), jnp.int32) + k
            a = plsc.load_gather(acc_v, [rows, col])
            u = plsc.load_gather(urow_v, [rows, col])
            return dot + a * u

        dot16 = lax.fori_loop(0, EMB, kstep, jnp.zeros((LANES,), jnp.float32))
        sl = pl.ds(g * LANES, LANES)
        out_v[sl] = (ub_v[sl] + tb_v[sl] * INV_HIST
                     + dot16 * INV_HIST + iq_v[sl])
        return carry

    lax.fori_loop(0, GROUPS, group, 0)
    pltpu.sync_copy(out_v, out_h.at[pl.ds(base, BPW)])


_sc_call = pl.kernel(
    _sc_body,
    out_type=jax.ShapeDtypeStruct((B,), jnp.float32),
    mesh=plsc.VectorSubcoreMesh(core_axis_name="c", subcore_axis_name="s"),
    scratch_types=[
        pltpu.VMEM((HIST, BPW), jnp.int32),
        pltpu.VMEM((BPW,), jnp.int32),
        pltpu.VMEM((BPW,), jnp.int32),
        pltpu.VMEM((BPW,), jnp.int32),
        pltpu.VMEM((BPW, EMB), jnp.float32),
        pltpu.VMEM((BPW, EMB), jnp.float32),
        pltpu.VMEM((BPW,), jnp.float32),
        pltpu.VMEM((BPW,), jnp.float32),
        pltpu.VMEM((BPW, EMB), jnp.float32),
        pltpu.VMEM((BPW, EMB), jnp.float32),
        pltpu.VMEM((BPW,), jnp.float32),
        pltpu.VMEM((BPW,), jnp.float32),
        pltpu.VMEM((BPW,), jnp.float32),
        pltpu.VMEM((BPW,), jnp.float32),
        pltpu.VMEM_SHARED((NS * BPW, EMB), jnp.float32),
        pltpu.SemaphoreType.DMA,
        pltpu.SemaphoreType.DMA,
        pltpu.SemaphoreType.DMA,
        pltpu.SemaphoreType.DMA,
        pltpu.SemaphoreType.DMA,
        pltpu.SemaphoreType.DMA,
    ],
    compiler_params=pltpu.CompilerParams(needs_layout_passes=False),
)


@jax.jit
def kernel(user, item, tag_index, user_e, tag_e, user_b, tag_b, item_q):
    tag_t = tag_index.astype(jnp.int32).T
    return _sc_call(user.astype(jnp.int32), item.astype(jnp.int32), tag_t,
                    user_e, tag_e, user_b.reshape(-1), tag_b.reshape(-1),
                    item_q.reshape(-1))
